# baseline (speedup 1.0000x reference)
import jax
import jax.numpy as jnp
from jax import lax
from jax.experimental import pallas as pl
from jax.experimental.pallas import tpu as pltpu
from jax.experimental.pallas import tpu_sc as plsc

B = 16384
L = 200
V = 1000
D_IN = 100
D_OUT = 21
DPAD = 24
VPAD = 1008
NC, NS = 2, 16
NW = NC * NS
BAGS_PER_W = B // NW
NG = BAGS_PER_W // 16
IDX_PER_W = BAGS_PER_W * L


def _table_body(emb_ref, w_ref, b_ref, out_ref):
    p = jnp.dot(emb_ref[...], w_ref[...].T,
                preferred_element_type=jnp.float32) * (1.0 / L)
    out_ref[...] = jnp.concatenate([p, b_ref[...]], axis=0)


def _fused_table(emb_weight, w_pad, b_rows):
    return pl.pallas_call(
        _table_body,
        out_shape=jax.ShapeDtypeStruct((VPAD, DPAD), jnp.float32),
    )(emb_weight, w_pad, b_rows)


def _sc_body(table_hbm, idx_hbm, out_hbm, table_v, idx_v, ob0, ob1,
             sem, osem0, osem1):
    wid = lax.axis_index("s") * NC + lax.axis_index("c")
    pltpu.sync_copy(table_hbm, table_v)
    pltpu.sync_copy(idx_hbm.at[pl.ds(wid * BAGS_PER_W, BAGS_PER_W)],
                    idx_v.at[pl.ds(0, BAGS_PER_W)])

    lanes = lax.iota(jnp.int32, 16)
    bias_row = jnp.full((16,), V, jnp.int32)
    cols = [jnp.full((16,), c, jnp.int32) for c in range(D_OUT)]

    def do_group(g, ob, osem):
        grows = lanes + g * 16
        vidx0 = plsc.load_gather(idx_v, [grows, jnp.zeros((16,), jnp.int32)])

        def body(l, carry):
            vidx = carry[0]
            a = list(carry[1:])
            vidx_next = plsc.load_gather(
                idx_v, [grows, jnp.full((16,), 1, jnp.int32) * jnp.minimum(l + 1, L - 1)])
            for c in range(D_OUT):
                a[c] = a[c] + plsc.load_gather(table_v, [vidx, cols[c]])
            return (vidx_next,) + tuple(a)

        init = tuple(plsc.load_gather(table_v, [bias_row, cols[c]])
                     for c in range(D_OUT))
        accs = lax.fori_loop(0, L, body, (vidx0,) + init)[1:]
        for c in range(D_OUT):
            plsc.store_scatter(ob, [lanes, cols[c]], accs[c])
        return pltpu.async_copy(
            ob, out_hbm.at[pl.ds(wid * BAGS_PER_W + g * 16, 16)], osem)

    @pl.loop(0, NG, step=2)
    def _(g):
        h0 = do_group(g, ob0, osem0)
        h1 = do_group(g + 1, ob1, osem1)
        h0.wait()
        h1.wait()


def kernel(input, emb_weight, lin_w, lin_b):
    idx = jnp.asarray(input, jnp.int32)
    w_pad = jnp.zeros((DPAD, D_IN), jnp.float32).at[:D_OUT].set(
        lin_w.astype(jnp.float32))
    b_rows = jnp.zeros((VPAD - V, DPAD), jnp.float32).at[:, :D_OUT].set(
        lin_b.astype(jnp.float32))
    table = _fused_table(emb_weight.astype(jnp.float32), w_pad, b_rows)

    mesh = plsc.VectorSubcoreMesh(core_axis_name="c", subcore_axis_name="s")
    bag_sum = pl.kernel(
        _sc_body,
        mesh=mesh,
        compiler_params=pltpu.CompilerParams(
            use_tc_tiling_on_sc=False, needs_layout_passes=False),
        out_type=jax.ShapeDtypeStruct((B, D_OUT), jnp.float32),
        scratch_types=[
            pltpu.VMEM((VPAD, DPAD), jnp.float32),
            pltpu.VMEM((BAGS_PER_W, L), jnp.int32),
            pltpu.VMEM((16, D_OUT), jnp.float32),
            pltpu.VMEM((16, D_OUT), jnp.float32),
            pltpu.SemaphoreType.DMA,
            pltpu.SemaphoreType.DMA,
            pltpu.SemaphoreType.DMA,
        ],
    )
    return bag_sum(table, idx)


# speedup vs baseline: 1.0042x; 1.0042x over previous
import jax
import jax.numpy as jnp
from jax import lax
from jax.experimental import pallas as pl
from jax.experimental.pallas import tpu as pltpu
from jax.experimental.pallas import tpu_sc as plsc

B = 16384
L = 200
V = 1000
D_IN = 100
D_OUT = 21
DPAD = 24
VPAD = 1008
NC, NS = 2, 16
NW = NC * NS
BAGS_PER_W = B // NW
NG = BAGS_PER_W // 16
IDX_PER_W = BAGS_PER_W * L


def _table_body(emb_ref, w_ref, b_ref, out_ref):
    p = jnp.dot(emb_ref[...], w_ref[...].T,
                preferred_element_type=jnp.float32) * (1.0 / L)
    out_ref[...] = jnp.concatenate([p, b_ref[...]], axis=0)


def _fused_table(emb_weight, w_pad, b_rows):
    return pl.pallas_call(
        _table_body,
        out_shape=jax.ShapeDtypeStruct((VPAD, DPAD), jnp.float32),
    )(emb_weight, w_pad, b_rows)


def _sc_body(table_hbm, idx_hbm, out_hbm, table_v, idx_v, ob0, ob1,
             sem, osem0, osem1):
    wid = lax.axis_index("s") * NC + lax.axis_index("c")
    pltpu.sync_copy(table_hbm, table_v)
    pltpu.sync_copy(idx_hbm.at[pl.ds(wid * IDX_PER_W, IDX_PER_W)],
                    idx_v.at[pl.ds(0, IDX_PER_W)])

    lanes = lax.iota(jnp.int32, 16)
    lane_off = lanes * L
    bias_row = jnp.full((16,), V, jnp.int32)
    cols = [jnp.full((16,), c, jnp.int32) for c in range(D_OUT)]

    def do_group(g, ob, osem):
        gbase = g * (16 * L)
        vidx0 = plsc.load_gather(idx_v, [lane_off + gbase])

        def body(l, carry):
            vidx = carry[0]
            a = list(carry[1:])
            vidx_next = plsc.load_gather(idx_v, [lane_off + (gbase + l + 1)])
            for c in range(D_OUT):
                a[c] = a[c] + plsc.load_gather(table_v, [vidx, cols[c]])
            return (vidx_next,) + tuple(a)

        init = tuple(plsc.load_gather(table_v, [bias_row, cols[c]])
                     for c in range(D_OUT))
        accs = lax.fori_loop(0, L, body, (vidx0,) + init)[1:]
        for c in range(D_OUT):
            plsc.store_scatter(ob, [lanes, cols[c]], accs[c])
        return pltpu.async_copy(
            ob, out_hbm.at[pl.ds(wid * BAGS_PER_W + g * 16, 16)], osem)

    @pl.loop(0, NG, step=2)
    def _(g):
        h0 = do_group(g, ob0, osem0)
        h1 = do_group(g + 1, ob1, osem1)
        h0.wait()
        h1.wait()


def kernel(input, emb_weight, lin_w, lin_b):
    idx = jnp.asarray(input, jnp.int32).reshape(-1)
    w_pad = jnp.zeros((DPAD, D_IN), jnp.float32).at[:D_OUT].set(
        lin_w.astype(jnp.float32))
    b_rows = jnp.zeros((VPAD - V, DPAD), jnp.float32).at[:, :D_OUT].set(
        lin_b.astype(jnp.float32))
    table = _fused_table(emb_weight.astype(jnp.float32), w_pad, b_rows)

    mesh = plsc.VectorSubcoreMesh(core_axis_name="c", subcore_axis_name="s")
    bag_sum = pl.kernel(
        _sc_body,
        mesh=mesh,
        compiler_params=pltpu.CompilerParams(
            use_tc_tiling_on_sc=False, needs_layout_passes=False),
        out_type=jax.ShapeDtypeStruct((B, D_OUT), jnp.float32),
        scratch_types=[
            pltpu.VMEM((VPAD, DPAD), jnp.float32),
            pltpu.VMEM((IDX_PER_W + 16,), jnp.int32),
            pltpu.VMEM((16, D_OUT), jnp.float32),
            pltpu.VMEM((16, D_OUT), jnp.float32),
            pltpu.SemaphoreType.DMA,
            pltpu.SemaphoreType.DMA,
            pltpu.SemaphoreType.DMA,
        ],
    )
    return bag_sum(table, idx)
